# baseline (speedup 1.0000x reference)
import functools

import jax
import jax.numpy as jnp
from jax import lax
from jax.experimental import pallas as pl
from jax.experimental.pallas import tpu as pltpu
from jax.experimental.pallas import tpu_sc as plsc

D = 32
NB = 50
NBP = 64
B = 4096
NW = 32
BPW = B // NW
PAIRS = BPW // 2
CIDX = 2 * NBP
CPC = 4
CROWS = CPC * CIDX
NCH = PAIRS // CPC
HALF = D // 2


RING = 4


_GDN = lax.GatherDimensionNumbers(
    offset_dims=(), collapsed_slice_dims=(0,), start_index_map=(0,))


def _lane(v, kk):
  idx = jnp.full((HALF, 1), kk, jnp.int32)
  return lax.gather(v, idx, _GDN, (1,),
                    mode=lax.GatherScatterMode.PROMISE_IN_BOUNDS)


def _sc_stage(uid, iid, snp, mask2, user_table, item_table):
  mesh = plsc.VectorSubcoreMesh(core_axis_name="c", subcore_axis_name="s")
  f32 = jnp.float32

  @functools.partial(
      pl.kernel,
      mesh=mesh,
      compiler_params=pltpu.CompilerParams(use_tc_tiling_on_sc=False),
      out_type=(
          jax.ShapeDtypeStruct((B, D), f32),
          jax.ShapeDtypeStruct((B, D), f32),
          jax.ShapeDtypeStruct((B, D), f32),
      ),
      scratch_types=[
          pltpu.VMEM((BPW,), jnp.int32),
          pltpu.VMEM((BPW,), jnp.int32),
          pltpu.VMEM((NCH, CROWS), jnp.int32),
          pltpu.VMEM((PAIRS * CIDX,), f32),
          pltpu.VMEM((BPW, D), f32),
          pltpu.VMEM((BPW, D), f32),
          pltpu.VMEM((RING, CROWS, D), f32),
          pltpu.VMEM((BPW, D), f32),
          pltpu.VMEM((BPW, D), f32),
          pltpu.SemaphoreType.DMA,
          pltpu.SemaphoreType.DMA,
      ] + [pltpu.SemaphoreType.DMA] * RING,
  )
  def k(uid_hbm, iid_hbm, snp_hbm, mask_hbm, utab_hbm, itab_hbm,
        pre0_hbm, pre1_hbm, item_hbm,
        idx_u, idx_i, idx_n, mask_v, urows, irows, nrows, pre0_v, pre1_v,
        sem_u, sem_i, *sems):
    w = lax.axis_index("s") * 2 + lax.axis_index("c")
    base = w * BPW

    pltpu.sync_copy(uid_hbm.at[pl.ds(base, BPW)], idx_u)
    pltpu.sync_copy(iid_hbm.at[pl.ds(base, BPW)], idx_i)
    pltpu.sync_copy(snp_hbm.at[pl.ds(w * NCH, NCH), :], idx_n)
    pltpu.sync_copy(mask_hbm.at[pl.ds(w * PAIRS * CIDX, PAIRS * CIDX)],
                    mask_v)

    cp_u = pltpu.async_copy(utab_hbm.at[idx_u], urows, sem_u)
    cp_i = pltpu.async_copy(itab_hbm.at[idx_i], irows, sem_i)

    for j in range(RING):
      pltpu.async_copy(utab_hbm.at[idx_n.at[j]], nrows.at[j], sems[j])
    cp_u.wait()

    inv = f32(1.0 / NB)
    zero = jnp.zeros((HALF,), f32)

    def chunk(c, j):
      pltpu.make_async_copy(utab_hbm.at[idx_n.at[0]], nrows.at[j],
                            sems[j]).wait()
      nb = nrows.at[j]
      c_off = c * CROWS
      for p in range(CPC):
        for bb in range(2):
          pq = p * CIDX + bb * NBP

          def tbody(t, accs, pq=pq, nb=nb, c_off=c_off):
            a00, a01, a10, a11 = accs
            qb = pq + t * HALF
            mvec = mask_v[pl.ds(c_off + qb, HALF)]
            for kk in range(HALF):
              m = _lane(mvec, kk)
              m2 = m * m
              r0 = nb[qb + kk, pl.ds(0, HALF)]
              r1 = nb[qb + kk, pl.ds(HALF, HALF)]
              a00 = a00 + m * r0
              a01 = a01 + m * r1
              a10 = a10 + m2 * r0
              a11 = a11 + m2 * r1
            return (a00, a01, a10, a11)

          a00, a01, a10, a11 = lax.fori_loop(0, NBP // HALF, tbody,
                                             (zero, zero, zero, zero))
          i = c * 2 * CPC + p * 2 + bb
          pre0_v[i, pl.ds(0, HALF)] = urows[i, pl.ds(0, HALF)] + a00 * inv
          pre0_v[i, pl.ds(HALF, HALF)] = (urows[i, pl.ds(HALF, HALF)]
                                          + a01 * inv)
          pre1_v[i, pl.ds(0, HALF)] = a10 * inv
          pre1_v[i, pl.ds(HALF, HALF)] = a11 * inv
      nc = c + RING

      @pl.when(nc < NCH)
      def _():
        pltpu.async_copy(utab_hbm.at[idx_n.at[nc]], nrows.at[j], sems[j])

    def body(g, carry):
      for j in range(RING):
        chunk(g * RING + j, j)
      return carry

    lax.fori_loop(0, NCH // RING, body, 0)

    pltpu.sync_copy(pre0_v, pre0_hbm.at[pl.ds(base, BPW), :])
    pltpu.sync_copy(pre1_v, pre1_hbm.at[pl.ds(base, BPW), :])
    cp_i.wait()
    pltpu.sync_copy(irows, item_hbm.at[pl.ds(base, BPW), :])

  return k(uid, iid, snp, mask2, user_table, item_table)


TCOL = 4096


def _tr_body(x_ref, o_ref):
  x = x_ref[...]
  r = lax.broadcasted_iota(jnp.int32, (D, D), 0)
  c = lax.broadcasted_iota(jnp.int32, (D, D), 1)
  eye = jnp.where(r == c, 1.0, 0.0)
  y = lax.dot_general(x, eye, (((0,), (0,)), ((), ())),
                      precision=lax.Precision.HIGHEST)
  y4 = y.reshape(TCOL // 4, 4, D)
  parts = [y4[:, q, :] for q in range(4)]
  o_ref[...] = jnp.concatenate(parts, axis=1)


def _tr_stage(tab_t, n_rows):
  g = pl.cdiv(n_rows, TCOL)
  return pl.pallas_call(
      _tr_body,
      grid=(g,),
      in_specs=[pl.BlockSpec((D, TCOL), lambda i: (0, i))],
      out_specs=pl.BlockSpec((TCOL // 4, 128), lambda i: (i, 0)),
      out_shape=jax.ShapeDtypeStruct((n_rows // 4, 128), jnp.float32),
  )(tab_t)


BM = 256


def _tc_body(pre0_ref, pre1_ref, item_ref, w0_ref, w1_ref, b_ref, out_ref):
  dn = (((1,), (1,)), ((), ()))
  u = pre0_ref[...]
  u = lax.dot_general(u, w0_ref[...], dn) + b_ref[0:1, :]
  u = jnp.maximum(u, 0.0)
  u = u + pre1_ref[...]
  u = lax.dot_general(u, w1_ref[...], dn) + b_ref[1:2, :]
  u = jnp.maximum(u, 0.0)
  out_ref[...] = lax.dot_general(u, item_ref[...], dn)


def _tc_stage(pre0, pre1, item_rows, w0, w1, bias):
  return pl.pallas_call(
      _tc_body,
      grid=(B // BM,),
      in_specs=[
          pl.BlockSpec((BM, D), lambda i: (i, 0)),
          pl.BlockSpec((BM, D), lambda i: (i, 0)),
          pl.BlockSpec((B, D), lambda i: (0, 0)),
          pl.BlockSpec((D, D), lambda i: (0, 0)),
          pl.BlockSpec((D, D), lambda i: (0, 0)),
          pl.BlockSpec((2, D), lambda i: (0, 0)),
      ],
      out_specs=pl.BlockSpec((BM, B), lambda i: (i, 0)),
      out_shape=jax.ShapeDtypeStruct((B, B), jnp.float32),
  )(pre0, pre1, item_rows, w0, w1, bias)


def kernel(user_ids, item_ids, social_neighbors, attention_mask,
           user_table, item_table, W, b):
  uid = user_ids.astype(jnp.int32)
  iid = item_ids.astype(jnp.int32)
  padidx = (jnp.arange(B * (NBP - NB), dtype=jnp.int32)
            % jnp.int32(1000000)).reshape(B, NBP - NB)
  snp = jnp.concatenate(
      [social_neighbors.astype(jnp.int32), padidx],
      axis=1).reshape(NW * NCH, CROWS)
  mask2 = jnp.pad(attention_mask, ((0, 0), (0, NBP - NB))).reshape(B * NBP)
  utab = _tr_stage(user_table.T, 1000000).reshape(1000000, D)
  itab = _tr_stage(item_table.T, 100000).reshape(100000, D)
  pre0, pre1, item_rows = _sc_stage(uid, iid, snp, mask2, utab, itab)
  return _tc_stage(pre0, pre1, item_rows, W[0], W[1], b)


# speedup vs baseline: 2.2971x; 2.2971x over previous
import functools

import jax
import jax.numpy as jnp
from jax import lax
from jax.experimental import pallas as pl
from jax.experimental.pallas import tpu as pltpu
from jax.experimental.pallas import tpu_sc as plsc

D = 32
NB = 50
NBP = 64
B = 4096
NW = 32
BPW = B // NW
PAIRS = BPW // 2
CIDX = 2 * NBP
CPC = 4
CROWS = CPC * CIDX
NCH = PAIRS // CPC
HALF = D // 2


RING = 4


_GDN = lax.GatherDimensionNumbers(
    offset_dims=(), collapsed_slice_dims=(0,), start_index_map=(0,))


def _lane(v, kk):
  idx = jnp.full((HALF, 1), kk, jnp.int32)
  return lax.gather(v, idx, _GDN, (1,),
                    mode=lax.GatherScatterMode.PROMISE_IN_BOUNDS)


def _sc_stage(uid, iid, snp, mask2, user_table, item_table):
  mesh = plsc.VectorSubcoreMesh(core_axis_name="c", subcore_axis_name="s")
  f32 = jnp.float32

  @functools.partial(
      pl.kernel,
      mesh=mesh,
      compiler_params=pltpu.CompilerParams(use_tc_tiling_on_sc=False),
      out_type=(
          jax.ShapeDtypeStruct((B, D), f32),
          jax.ShapeDtypeStruct((B, D), f32),
          jax.ShapeDtypeStruct((B, D), f32),
      ),
      scratch_types=[
          pltpu.VMEM((BPW,), jnp.int32),
          pltpu.VMEM((BPW,), jnp.int32),
          pltpu.VMEM((NCH, CROWS), jnp.int32),
          pltpu.VMEM((PAIRS * CIDX,), f32),
          pltpu.VMEM((BPW, D), f32),
          pltpu.VMEM((BPW, D), f32),
          pltpu.VMEM((RING, CROWS, D), f32),
          pltpu.VMEM((BPW, D), f32),
          pltpu.VMEM((BPW, D), f32),
          pltpu.SemaphoreType.DMA,
          pltpu.SemaphoreType.DMA,
      ] + [pltpu.SemaphoreType.DMA] * RING,
  )
  def k(uid_hbm, iid_hbm, snp_hbm, mask_hbm, utab_hbm, itab_hbm,
        pre0_hbm, pre1_hbm, item_hbm,
        idx_u, idx_i, idx_n, mask_v, urows, irows, nrows, pre0_v, pre1_v,
        sem_u, sem_i, *sems):
    w = lax.axis_index("s") * 2 + lax.axis_index("c")
    base = w * BPW

    pltpu.sync_copy(uid_hbm.at[pl.ds(base, BPW)], idx_u)
    pltpu.sync_copy(iid_hbm.at[pl.ds(base, BPW)], idx_i)
    pltpu.sync_copy(snp_hbm.at[pl.ds(w * NCH, NCH), :], idx_n)
    pltpu.sync_copy(mask_hbm.at[pl.ds(w * PAIRS * CIDX, PAIRS * CIDX)],
                    mask_v)

    cp_u = pltpu.async_copy(utab_hbm.at[idx_u], urows, sem_u)
    cp_i = pltpu.async_copy(itab_hbm.at[idx_i], irows, sem_i)

    for j in range(RING):
      pltpu.async_copy(utab_hbm.at[idx_n.at[j]], nrows.at[j], sems[j])
    cp_u.wait()

    inv = f32(1.0 / NB)
    zero = jnp.zeros((HALF,), f32)

    def chunk(c, j):
      pltpu.make_async_copy(utab_hbm.at[idx_n.at[0]], nrows.at[j],
                            sems[j]).wait()
      nb = nrows.at[j]
      c_off = c * CROWS
      for p in range(CPC):
        for bb in range(2):
          pq = p * CIDX + bb * NBP

          def tbody(t, accs, pq=pq, nb=nb, c_off=c_off):
            a00, a01, a10, a11 = accs
            qb = pq + t * HALF
            mvec = mask_v[pl.ds(c_off + qb, HALF)]
            for kk in range(HALF):
              m = _lane(mvec, kk)
              m2 = m * m
              r0 = nb[qb + kk, pl.ds(0, HALF)]
              r1 = nb[qb + kk, pl.ds(HALF, HALF)]
              a00 = a00 + m * r0
              a01 = a01 + m * r1
              a10 = a10 + m2 * r0
              a11 = a11 + m2 * r1
            return (a00, a01, a10, a11)

          a00, a01, a10, a11 = lax.fori_loop(0, NBP // HALF, tbody,
                                             (zero, zero, zero, zero))
          i = c * 2 * CPC + p * 2 + bb
          pre0_v[i, pl.ds(0, HALF)] = urows[i, pl.ds(0, HALF)] + a00 * inv
          pre0_v[i, pl.ds(HALF, HALF)] = (urows[i, pl.ds(HALF, HALF)]
                                          + a01 * inv)
          pre1_v[i, pl.ds(0, HALF)] = a10 * inv
          pre1_v[i, pl.ds(HALF, HALF)] = a11 * inv
      nc = c + RING

      @pl.when(nc < NCH)
      def _():
        pltpu.async_copy(utab_hbm.at[idx_n.at[nc]], nrows.at[j], sems[j])

    def body(g, carry):
      for j in range(RING):
        chunk(g * RING + j, j)
      return carry

    lax.fori_loop(0, NCH // RING, body, 0)

    pltpu.sync_copy(pre0_v, pre0_hbm.at[pl.ds(base, BPW), :])
    pltpu.sync_copy(pre1_v, pre1_hbm.at[pl.ds(base, BPW), :])
    cp_i.wait()
    pltpu.sync_copy(irows, item_hbm.at[pl.ds(base, BPW), :])

  return k(uid, iid, snp, mask2, user_table, item_table)


TBLK = 8192
TSUB = TBLK // 4


def _tr_body(x_ref, o_ref):
  y = x_ref[...].T
  parts = [y[q * TSUB:(q + 1) * TSUB, :] for q in range(4)]
  o_ref[...] = jnp.concatenate(parts, axis=1)


def _tr_stage(tab_t, n_rows):
  g = pl.cdiv(n_rows, TBLK)
  return pl.pallas_call(
      _tr_body,
      grid=(g,),
      in_specs=[pl.BlockSpec((D, TBLK), lambda i: (0, i))],
      out_specs=pl.BlockSpec((TSUB, 128), lambda i: (i, 0)),
      out_shape=jax.ShapeDtypeStruct((g * TSUB, 128), jnp.float32),
  )(tab_t)


def _perm(idx):
  l = idx % TBLK
  return (idx - l) + 4 * (l % TSUB) + l // TSUB


BM = 256


def _tc_body(pre0_ref, pre1_ref, item_ref, w0_ref, w1_ref, b_ref, out_ref):
  dn = (((1,), (1,)), ((), ()))
  u = pre0_ref[...]
  u = lax.dot_general(u, w0_ref[...], dn) + b_ref[0:1, :]
  u = jnp.maximum(u, 0.0)
  u = u + pre1_ref[...]
  u = lax.dot_general(u, w1_ref[...], dn) + b_ref[1:2, :]
  u = jnp.maximum(u, 0.0)
  out_ref[...] = lax.dot_general(u, item_ref[...], dn)


def _tc_stage(pre0, pre1, item_rows, w0, w1, bias):
  return pl.pallas_call(
      _tc_body,
      grid=(B // BM,),
      in_specs=[
          pl.BlockSpec((BM, D), lambda i: (i, 0)),
          pl.BlockSpec((BM, D), lambda i: (i, 0)),
          pl.BlockSpec((B, D), lambda i: (0, 0)),
          pl.BlockSpec((D, D), lambda i: (0, 0)),
          pl.BlockSpec((D, D), lambda i: (0, 0)),
          pl.BlockSpec((2, D), lambda i: (0, 0)),
      ],
      out_specs=pl.BlockSpec((BM, B), lambda i: (i, 0)),
      out_shape=jax.ShapeDtypeStruct((B, B), jnp.float32),
  )(pre0, pre1, item_rows, w0, w1, bias)


def kernel(user_ids, item_ids, social_neighbors, attention_mask,
           user_table, item_table, W, b):
  uid = _perm(user_ids.astype(jnp.int32))
  iid = _perm(item_ids.astype(jnp.int32))
  padidx = (jnp.arange(B * (NBP - NB), dtype=jnp.int32)
            % jnp.int32(1000000)).reshape(B, NBP - NB)
  snp = _perm(
      jnp.concatenate([social_neighbors.astype(jnp.int32), padidx],
                      axis=1)).reshape(NW * NCH, CROWS)
  mask2 = jnp.pad(attention_mask, ((0, 0), (0, NBP - NB))).reshape(B * NBP)
  utab = _tr_stage(user_table.T, 1000000)
  utab = utab.reshape(utab.shape[0] * 4, D)
  itab = _tr_stage(item_table.T, 100000)
  itab = itab.reshape(itab.shape[0] * 4, D)
  pre0, pre1, item_rows = _sc_stage(uid, iid, snp, mask2, utab, itab)
  return _tc_stage(pre0, pre1, item_rows, W[0], W[1], b)
